# baseline (speedup 1.0000x reference)
import functools

import jax
import jax.numpy as jnp
from jax import lax
from jax.experimental import pallas as pl
from jax.experimental.pallas import tpu as pltpu
from jax.experimental.pallas import tpu_sc as plsc

_B = 4096
_L = 200
_D = 64
_MAX_LEN = 256
_N = _B * _L
_NC = 2
_NS = 16
_NW = _NC * _NS
_LANES = _B // _NW
_ROWS_PER_W = _N // _NW
_LB = 2
_BLKS = _L // _LB
_NBUF = 4


@jax.jit
def _pos_gather(idx_flat, pet_flat):
    mesh = plsc.VectorSubcoreMesh(core_axis_name="c", subcore_axis_name="s")

    @functools.partial(
        pl.kernel,
        mesh=mesh,
        compiler_params=pltpu.CompilerParams(needs_layout_passes=False),
        out_type=jax.ShapeDtypeStruct((_L, _D, _B), jnp.float32),
        scratch_types=[
            pltpu.VMEM((_ROWS_PER_W,), jnp.int32),
            pltpu.VMEM((_D * _MAX_LEN,), jnp.float32),
            pltpu.VMEM((_NBUF, _LB, _D, _LANES), jnp.float32),
            pltpu.SemaphoreType.DMA,
        ],
    )
    def body(idx_hbm, pet_hbm, out_hbm, idx_v, table_v, slab_v, osem):
        wid = lax.axis_index("s") * _NC + lax.axis_index("c")
        pltpu.sync_copy(idx_hbm.at[pl.ds(wid * _ROWS_PER_W, _ROWS_PER_W)],
                        idx_v)
        pltpu.sync_copy(pet_hbm, table_v)

        bstride = lax.iota(jnp.int32, 16) * _L

        def out_slice(blk):
            return out_hbm.at[pl.ds(blk * _LB, _LB), slice(None),
                              pl.ds(wid * _LANES, _LANES)]

        def start_out(blk):
            pltpu.async_copy(slab_v.at[lax.rem(blk, _NBUF)], out_slice(blk),
                             osem)

        def wait_out(blk):
            pltpu.make_async_copy(slab_v.at[lax.rem(blk, _NBUF)],
                                  out_slice(blk), osem).wait()

        def compute_block(blk, slot):
            for lb in range(_LB):
                l = blk * _LB + lb

                def per_group(g, c):
                    flat = bstride + (g * (16 * _L) + l)
                    idx16 = plsc.load_gather(idx_v, [flat])
                    a = jnp.minimum(idx16, _MAX_LEN - 1)
                    for d in range(_D):
                        vals = plsc.load_gather(table_v, [a + d * _MAX_LEN])
                        slab_v[slot, lb, d, pl.ds(g * 16, 16)] = vals
                    return c

                lax.fori_loop(0, _LANES // 16, per_group, 0)

        def step(blk, carry):
            slot = lax.rem(blk, _NBUF)

            @pl.when(blk >= _NBUF)
            def _():
                wait_out(blk - _NBUF)

            compute_block(blk, slot)
            start_out(blk)
            return carry

        lax.fori_loop(0, _BLKS, step, 0)

        for blk in range(_BLKS - _NBUF, _BLKS):
            wait_out(blk)

    return body(idx_flat, pet_flat)


def kernel(idx_or_len, pe, device=0):
    idx_flat = idx_or_len.astype(jnp.int32).reshape(_N)
    pet_flat = pe.astype(jnp.float32).T.reshape(_D * _MAX_LEN)
    out_t = _pos_gather(idx_flat, pet_flat)
    return jnp.transpose(out_t, (2, 0, 1))


# speedup vs baseline: 2.2031x; 2.2031x over previous
import functools

import jax
import jax.numpy as jnp
from jax import lax
from jax.experimental import pallas as pl
from jax.experimental.pallas import tpu as pltpu
from jax.experimental.pallas import tpu_sc as plsc

_B = 4096
_L = 200
_D = 64
_MAX_LEN = 256
_N = _B * _L
_NC = 2
_NS = 16
_NW = _NC * _NS
_LANES = _B // _NW
_ROWS_PER_W = _N // _NW
_LB = 2
_BLKS = _L // _LB
_NBUF = 4


@jax.jit
def _pos_gather(idx_flat, pet_flat):
    mesh = plsc.VectorSubcoreMesh(core_axis_name="c", subcore_axis_name="s")

    @functools.partial(
        pl.kernel,
        mesh=mesh,
        compiler_params=pltpu.CompilerParams(needs_layout_passes=False),
        out_type=jax.ShapeDtypeStruct((_L, _D, _B), jnp.float32),
        scratch_types=[
            pltpu.VMEM((_ROWS_PER_W,), jnp.int32),
            pltpu.VMEM((_D * _MAX_LEN,), jnp.float32),
            pltpu.VMEM((_NBUF, _LB, _D, _LANES), jnp.float32),
            pltpu.SemaphoreType.DMA,
        ],
    )
    def body(idx_hbm, pet_hbm, out_hbm, idx_v, table_v, slab_v, osem):
        wid = lax.axis_index("s") * _NC + lax.axis_index("c")
        pltpu.sync_copy(idx_hbm.at[pl.ds(wid * _ROWS_PER_W, _ROWS_PER_W)],
                        idx_v)
        pltpu.sync_copy(pet_hbm, table_v)

        bstride = lax.iota(jnp.int32, 16) * _L

        def out_slice(blk):
            return out_hbm.at[pl.ds(blk * _LB, _LB), slice(None),
                              pl.ds(wid * _LANES, _LANES)]

        def start_out(blk):
            pltpu.async_copy(slab_v.at[lax.rem(blk, _NBUF)], out_slice(blk),
                             osem)

        def wait_out(blk):
            pltpu.make_async_copy(slab_v.at[lax.rem(blk, _NBUF)],
                                  out_slice(blk), osem).wait()

        def compute_block(blk, slot):
            for lb in range(_LB):
                l = blk * _LB + lb

                def per_group(g, c):
                    flat = bstride + (g * (16 * _L) + l)
                    idx16 = plsc.load_gather(idx_v, [flat])
                    a = jnp.minimum(idx16, _MAX_LEN - 1)

                    @plsc.parallel_loop(0, _D, unroll=8)
                    def _(d):
                        vals = plsc.load_gather(table_v, [a + d * _MAX_LEN])
                        slab_v[slot, lb, d, pl.ds(g * 16, 16)] = vals

                    return c

                lax.fori_loop(0, _LANES // 16, per_group, 0)

        def step(blk, carry):
            slot = lax.rem(blk, _NBUF)

            @pl.when(blk >= _NBUF)
            def _():
                wait_out(blk - _NBUF)

            compute_block(blk, slot)
            start_out(blk)
            return carry

        lax.fori_loop(0, _BLKS, step, 0)

        for blk in range(_BLKS - _NBUF, _BLKS):
            wait_out(blk)

    return body(idx_flat, pet_flat)


def kernel(idx_or_len, pe, device=0):
    idx_flat = idx_or_len.astype(jnp.int32).reshape(_N)
    pet_flat = pe.astype(jnp.float32).T.reshape(_D * _MAX_LEN)
    out_t = _pos_gather(idx_flat, pet_flat)
    return jnp.transpose(out_t, (2, 0, 1))
